# Initial kernel scaffold; baseline (speedup 1.0000x reference)
import jax
import jax.numpy as jnp
from jax.experimental import pallas as pl


def kernel(inputs, targets, emb_table, W, b):
    raise NotImplementedError("write your pallas kernel here")


# speedup vs baseline: 1.3152x; 1.3152x over previous
import dataclasses
import functools

import jax
import jax.numpy as jnp
from jax import lax
from jax.experimental import pallas as pl
from jax.experimental.pallas import tpu as pltpu
from jax.experimental.pallas import tpu_sc as plsc

V = 1000
NE = 128
NTOK = 32 * 2048
NWORK = 32
TPW = NTOK // NWORK
WIN = 32
NWIN = TPW // WIN
L = 16

ROWBLK = 40


VP = 1024


def _table_body(emb_ref, w_ref, b_ref, fused_ref):
    logits = jnp.dot(emb_ref[...], w_ref[...],
                     preferred_element_type=jnp.float32) + b_ref[...]
    m = jnp.max(logits, axis=1, keepdims=True)
    lse = m + jnp.log(jnp.sum(jnp.exp(logits - m), axis=1, keepdims=True))
    pad = jnp.zeros((ROWBLK, VP - V - 1), jnp.float32)
    fused_ref[...] = jnp.concatenate([logits, lse, pad], axis=1)


def _make_table(emb_table, W, b):
    return pl.pallas_call(
        _table_body,
        grid=(V // ROWBLK,),
        in_specs=[
            pl.BlockSpec((ROWBLK, NE), lambda i: (i, 0)),
            pl.BlockSpec((NE, V), lambda i: (0, 0)),
            pl.BlockSpec((1, V), lambda i: (0, 0)),
        ],
        out_specs=[
            pl.BlockSpec((ROWBLK, VP), lambda i: (i, 0)),
        ],
        out_shape=[
            jax.ShapeDtypeStruct((V, VP), jnp.float32),
        ],
    )(emb_table, W, b.reshape(1, V))[0]


def _sc_gather(fused, idx, tgt):
    mesh = plsc.VectorSubcoreMesh(core_axis_name="c", subcore_axis_name="s")
    cp = pltpu.CompilerParams(needs_layout_passes=False,
                              use_tc_tiling_on_sc=False)

    @functools.partial(
        pl.kernel,
        compiler_params=cp,
        out_type=(
            jax.ShapeDtypeStruct((NTOK, V), jnp.float32),
            jax.ShapeDtypeStruct((NWORK, L), jnp.float32),
        ),
        mesh=mesh,
        scratch_types=[
            pltpu.VMEM((TPW,), jnp.int32),
            pltpu.VMEM((TPW,), jnp.int32),
            pltpu.VMEM((WIN, VP), jnp.float32),
            pltpu.VMEM((L,), jnp.float32),
            pltpu.SemaphoreType.DMA,
        ],
    )
    def k(fused_hbm, idx_hbm, tgt_hbm, out_hbm, part_hbm,
          idx_v, tgt_v, rows, acc_v, gsem):
        wid = lax.axis_index("s") * 2 + lax.axis_index("c")
        base = wid * TPW
        pltpu.sync_copy(idx_hbm.at[pl.ds(base, TPW)], idx_v)
        pltpu.sync_copy(tgt_hbm.at[pl.ds(base, TPW)], tgt_v)
        acc_v[...] = jnp.zeros((L,), jnp.float32)

        @pl.loop(0, NWIN)
        def _(w):
            woff = w * WIN
            pltpu.async_copy(fused_hbm.at[idx_v.at[pl.ds(woff, WIN)]],
                             rows, gsem).wait()
            lse_col = jnp.full((L,), V, jnp.int32)
            for g in range(WIN // L):
                toff = woff + g * L
                tg = tgt_v[pl.ds(toff, L)]
                rowi = lax.iota(jnp.int32, L) + (g * L)
                lsev = plsc.load_gather(rows, [rowi, lse_col])
                logit_t = plsc.load_gather(rows, [rowi, tg])
                acc_v[...] = acc_v[...] + (lsev - logit_t)
            pltpu.sync_copy(rows.at[:, pl.ds(0, V)],
                            out_hbm.at[pl.ds(base + woff, WIN)])

        pltpu.sync_copy(acc_v, part_hbm.at[wid])

    return k(fused, idx, tgt)


def _loss_body(part_ref, out_ref):
    out_ref[...] = jnp.sum(part_ref[...], keepdims=True).reshape(1, 1) * (
        1.0 / NTOK)


def _reduce_loss(parts):
    return pl.pallas_call(
        _loss_body,
        out_shape=jax.ShapeDtypeStruct((1, 1), jnp.float32),
    )(parts)


def kernel(inputs, targets, emb_table, W, b):
    idx = inputs.reshape(-1).astype(jnp.int32)
    tgt = targets.reshape(-1).astype(jnp.int32)
    fused = _make_table(emb_table, W, b)
    logits, parts = _sc_gather(fused, idx, tgt)
    loss = _reduce_loss(parts)[0, 0]
    return (logits, loss)


# speedup vs baseline: 2.1240x; 1.6150x over previous
import dataclasses
import functools

import jax
import jax.numpy as jnp
from jax import lax
from jax.experimental import pallas as pl
from jax.experimental.pallas import tpu as pltpu
from jax.experimental.pallas import tpu_sc as plsc

V = 1000
NE = 128
NTOK = 32 * 2048
NWORK = 32
TPW = NTOK // NWORK
WIN = 32
NWIN = TPW // WIN
L = 16

ROWBLK = 40


VP = 1024


def _table_body(emb_ref, w_ref, b_ref, fused_ref):
    logits = jnp.dot(emb_ref[...], w_ref[...],
                     preferred_element_type=jnp.float32) + b_ref[...]
    m = jnp.max(logits, axis=1, keepdims=True)
    lse = m + jnp.log(jnp.sum(jnp.exp(logits - m), axis=1, keepdims=True))
    pad = jnp.zeros((ROWBLK, VP - V - 1), jnp.float32)
    fused_ref[...] = jnp.concatenate([logits, lse, pad], axis=1)


def _make_table(emb_table, W, b):
    return pl.pallas_call(
        _table_body,
        grid=(V // ROWBLK,),
        in_specs=[
            pl.BlockSpec((ROWBLK, NE), lambda i: (i, 0)),
            pl.BlockSpec((NE, V), lambda i: (0, 0)),
            pl.BlockSpec((1, V), lambda i: (0, 0)),
        ],
        out_specs=[
            pl.BlockSpec((ROWBLK, VP), lambda i: (i, 0)),
        ],
        out_shape=[
            jax.ShapeDtypeStruct((V, VP), jnp.float32),
        ],
    )(emb_table, W, b.reshape(1, V))[0]


def _sc_gather(fused, idx, tgt):
    mesh = plsc.VectorSubcoreMesh(core_axis_name="c", subcore_axis_name="s")
    cp = pltpu.CompilerParams(needs_layout_passes=False,
                              use_tc_tiling_on_sc=True)

    @functools.partial(
        pl.kernel,
        compiler_params=cp,
        out_type=(
            jax.ShapeDtypeStruct((NTOK, VP), jnp.float32),
            jax.ShapeDtypeStruct((NWORK, L), jnp.float32),
        ),
        mesh=mesh,
        scratch_types=[
            pltpu.VMEM((TPW,), jnp.int32),
            pltpu.VMEM((TPW,), jnp.int32),
            pltpu.VMEM((WIN, VP), jnp.float32),
            pltpu.VMEM((L,), jnp.float32),
            pltpu.SemaphoreType.DMA,
        ],
    )
    def k(fused_hbm, idx_hbm, tgt_hbm, out_hbm, part_hbm,
          idx_v, tgt_v, rows, acc_v, gsem):
        wid = lax.axis_index("s") * 2 + lax.axis_index("c")
        base = wid * TPW
        pltpu.sync_copy(idx_hbm.at[pl.ds(base, TPW)], idx_v)
        pltpu.sync_copy(tgt_hbm.at[pl.ds(base, TPW)], tgt_v)
        acc_v[...] = jnp.zeros((L,), jnp.float32)

        @pl.loop(0, NWIN)
        def _(w):
            woff = w * WIN
            pltpu.async_copy(fused_hbm.at[idx_v.at[pl.ds(woff, WIN)]],
                             rows, gsem).wait()
            lse_col = jnp.full((L,), V, jnp.int32)
            for g in range(WIN // L):
                toff = woff + g * L
                tg = tgt_v[pl.ds(toff, L)]
                rowi = lax.iota(jnp.int32, L) + (g * L)
                lsev = plsc.load_gather(rows, [rowi, lse_col])
                logit_t = plsc.load_gather(rows, [rowi, tg])
                acc_v[...] = acc_v[...] + (lsev - logit_t)
            pltpu.sync_copy(rows, out_hbm.at[pl.ds(base + woff, WIN)])

        pltpu.sync_copy(acc_v, part_hbm.at[wid])

    return k(fused, idx, tgt)


def _loss_body(part_ref, out_ref):
    out_ref[...] = jnp.sum(part_ref[...], keepdims=True).reshape(1, 1) * (
        1.0 / NTOK)


def _reduce_loss(parts):
    return pl.pallas_call(
        _loss_body,
        out_shape=jax.ShapeDtypeStruct((1, 1), jnp.float32),
    )(parts)


def kernel(inputs, targets, emb_table, W, b):
    idx = inputs.reshape(-1).astype(jnp.int32)
    tgt = targets.reshape(-1).astype(jnp.int32)
    fused = _make_table(emb_table, W, b)
    logits_pad, parts = _sc_gather(fused, idx, tgt)
    loss = _reduce_loss(parts)[0, 0]
    return (logits_pad[:, :V], loss)
